# baseline (speedup 1.0000x reference)
import functools
import math

import jax
import jax.numpy as jnp
from jax import lax
from jax.experimental import pallas as pl
from jax.experimental.pallas import tpu as pltpu
from jax.experimental.pallas import tpu_sc as plsc

B, S, D = 1, 2048, 768
NH, HD = 12, 64
E, TOPK = 8, 2
DFF = 4 * D
TT = 256
NT = S // TT
FT = 256
G_MAX = (S * TOPK) // FT + E
A_PAD = G_MAX * FT
NW = 32
CHUNK = S // NW
NEG_INF = float('-inf')


def _ln(x, g, b, eps=1e-5):
    mu = jnp.mean(x, axis=-1, keepdims=True)
    var = jnp.mean((x - mu) ** 2, axis=-1, keepdims=True)
    return (x - mu) * jax.lax.rsqrt(var + eps) * g + b


def _ln1_body(x_ref, g_ref, b_ref, o_ref):
    o_ref[...] = _ln(x_ref[...], g_ref[...], b_ref[...])


def _ln1(x2d, g, b):
    return pl.pallas_call(
        _ln1_body,
        grid=(NT,),
        in_specs=[
            pl.BlockSpec((TT, D), lambda t: (t, 0)),
            pl.BlockSpec((1, D), lambda t: (0, 0)),
            pl.BlockSpec((1, D), lambda t: (0, 0)),
        ],
        out_specs=pl.BlockSpec((TT, D), lambda t: (t, 0)),
        out_shape=jax.ShapeDtypeStruct((S, D), jnp.float32),
    )(x2d, g.reshape(1, D), b.reshape(1, D))


def _attn_body(h1_ref, wqkv_ref, bqkv_ref, o_ref):
    h1 = h1_ref[...].astype(jnp.bfloat16)
    qkv = jnp.dot(h1, wqkv_ref[0],
                  preferred_element_type=jnp.float32) + bqkv_ref[0]
    outs = []
    for j in range(2):
        base = j * 3 * HD
        q = qkv[:, base:base + HD] * jnp.float32(1.0 / math.sqrt(HD))
        k = qkv[:, base + HD:base + 2 * HD]
        v = qkv[:, base + 2 * HD:base + 3 * HD]
        s = jax.lax.dot_general(q.astype(jnp.bfloat16),
                                k.astype(jnp.bfloat16),
                                (((1,), (1,)), ((), ())),
                                preferred_element_type=jnp.float32)
        m = jnp.max(s, axis=1, keepdims=True)
        p = jnp.exp((s - m).astype(jnp.bfloat16))
        l = jnp.sum(p.astype(jnp.float32), axis=1, keepdims=True)
        ctx = jnp.dot(p, v.astype(jnp.bfloat16),
                      preferred_element_type=jnp.float32)
        outs.append(ctx / l)
    o_ref[...] = jnp.concatenate(outs, axis=1)


def _attention(h1, wqkv, bqkv):
    return pl.pallas_call(
        _attn_body,
        grid=(NH // 2,),
        in_specs=[
            pl.BlockSpec((S, D), lambda h: (0, 0)),
            pl.BlockSpec((1, D, 6 * HD), lambda h: (h, 0, 0)),
            pl.BlockSpec((1, 1, 6 * HD), lambda h: (h, 0, 0)),
        ],
        out_specs=pl.BlockSpec((S, 2 * HD), lambda h: (0, h)),
        out_shape=jax.ShapeDtypeStruct((S, D), jnp.float32),
    )(h1, wqkv, bqkv)


def _router_body(x_ref, ctx_ref, wo_ref, bo_ref, g2_ref, b2_ref,
                 rw_ref, rb_ref, nw_ref, nb_ref, nz_ref,
                 xa_ref, h2_ref, i0_ref, i1_ref, r0_ref, r1_ref,
                 g0_ref, g1_ref, cnt_ref, carry_ref):
    t = pl.program_id(0)
    sa = jnp.dot(ctx_ref[...].astype(jnp.bfloat16),
                 wo_ref[...].astype(jnp.bfloat16),
                 preferred_element_type=jnp.float32) + bo_ref[...]
    xa = x_ref[...] + sa
    xa_ref[...] = xa
    h2 = _ln(xa, g2_ref[...], b2_ref[...])
    h2_ref[...] = h2
    h2b = h2.astype(jnp.bfloat16)
    logits = jnp.dot(h2b, rw_ref[...].astype(jnp.bfloat16),
                     preferred_element_type=jnp.float32) + rb_ref[...]
    nl = jnp.dot(h2b, nw_ref[...].astype(jnp.bfloat16),
                 preferred_element_type=jnp.float32) + nb_ref[...]
    sp = jnp.maximum(nl, 0.0) + jnp.log1p(jnp.exp(-jnp.abs(nl)))
    lg = logits + nz_ref[...] * sp
    cols = jax.lax.broadcasted_iota(jnp.int32, (TT, E), 1)
    v0 = jnp.max(lg, axis=1, keepdims=True)
    i0 = jnp.argmax(lg, axis=1).reshape(TT, 1)
    masked = jnp.where(cols == i0, NEG_INF, lg)
    v1 = jnp.max(masked, axis=1, keepdims=True)
    i1 = jnp.argmax(masked, axis=1).reshape(TT, 1)
    ex = jnp.exp(v1 - v0)
    den = 1.0 + ex
    i0_ref[...] = i0
    i1_ref[...] = i1
    g0_ref[...] = 1.0 / den
    g1_ref[...] = ex / den

    oh0 = (cols == i0).astype(jnp.float32)
    oh1 = (cols == i1).astype(jnp.float32)
    m = oh0 + oh1
    rows_i = jax.lax.broadcasted_iota(jnp.int32, (TT, TT), 0)
    cols_i = jax.lax.broadcasted_iota(jnp.int32, (TT, TT), 1)
    tri = (cols_i < rows_i).astype(jnp.bfloat16)
    csum = jnp.dot(tri, m.astype(jnp.bfloat16),
                   preferred_element_type=jnp.float32)

    @pl.when(t == 0)
    def _():
        carry_ref[...] = jnp.zeros((1, E), jnp.float32)

    carry = carry_ref[...]
    rank = carry + csum
    r0_ref[...] = jnp.sum(oh0 * rank, axis=1, keepdims=True).astype(jnp.int32)
    r1_ref[...] = jnp.sum(oh1 * rank, axis=1, keepdims=True).astype(jnp.int32)
    new_carry = carry + jnp.sum(m, axis=0, keepdims=True)
    carry_ref[...] = new_carry
    cnt_ref[...] = new_carry.astype(jnp.int32)


def _router(x2d, ctx, wo_r, bo, g2, b2, rw, rb, nw, nb, nz):
    fixed = lambda shape: pl.BlockSpec(shape, lambda t: tuple(0 for _ in shape))
    col = lambda: pl.BlockSpec((TT, 1), lambda t: (t, 0))
    return pl.pallas_call(
        _router_body,
        grid=(NT,),
        in_specs=[
            pl.BlockSpec((TT, D), lambda t: (t, 0)),
            pl.BlockSpec((TT, D), lambda t: (t, 0)),
            fixed((D, D)), fixed((1, D)),
            fixed((1, D)), fixed((1, D)),
            fixed((D, E)), fixed((1, E)), fixed((D, E)), fixed((1, E)),
            pl.BlockSpec((TT, E), lambda t: (t, 0)),
        ],
        out_specs=[
            pl.BlockSpec((TT, D), lambda t: (t, 0)),
            pl.BlockSpec((TT, D), lambda t: (t, 0)),
            col(), col(), col(), col(), col(), col(),
            pl.BlockSpec((1, E), lambda t: (0, 0)),
        ],
        out_shape=[
            jax.ShapeDtypeStruct((S, D), jnp.float32),
            jax.ShapeDtypeStruct((S, D), jnp.float32),
            jax.ShapeDtypeStruct((S, 1), jnp.int32),
            jax.ShapeDtypeStruct((S, 1), jnp.int32),
            jax.ShapeDtypeStruct((S, 1), jnp.int32),
            jax.ShapeDtypeStruct((S, 1), jnp.int32),
            jax.ShapeDtypeStruct((S, 1), jnp.float32),
            jax.ShapeDtypeStruct((S, 1), jnp.float32),
            jax.ShapeDtypeStruct((1, E), jnp.int32),
        ],
        scratch_shapes=[pltpu.VMEM((1, E), jnp.float32)],
        compiler_params=pltpu.CompilerParams(
            dimension_semantics=("arbitrary",)),
    )(x2d, ctx, wo_r, bo.reshape(1, D), g2.reshape(1, D), b2.reshape(1, D),
      rw, rb.reshape(1, E), nw, nb.reshape(1, E), nz)


def _sched_body(i0_ref, i1_ref, r0_ref, r1_ref, cnt_ref,
                d0_ref, d1_ref, be_ref, nb_ref):
    cnt = cnt_ref[...]
    nbe = (cnt + (FT - 1)) // FT
    ecols = jax.lax.broadcasted_iota(jnp.int32, (E, E), 1)
    erows = jax.lax.broadcasted_iota(jnp.int32, (E, E), 0)
    triu = (erows <= ecols).astype(jnp.bfloat16)
    bcum = jnp.dot(nbe.astype(jnp.bfloat16), triu,
                   preferred_element_type=jnp.float32).astype(jnp.int32)
    off = (bcum - nbe) * FT
    cols = jax.lax.broadcasted_iota(jnp.int32, (S, E), 1)
    oh0 = (cols == i0_ref[...]).astype(jnp.float32)
    oh1 = (cols == i1_ref[...]).astype(jnp.float32)
    offf = off.astype(jnp.float32)
    d0 = r0_ref[...] + jnp.sum(oh0 * offf, axis=1,
                               keepdims=True).astype(jnp.int32)
    d1 = r1_ref[...] + jnp.sum(oh1 * offf, axis=1,
                               keepdims=True).astype(jnp.int32)
    d0_ref[...] = jnp.clip(d0, 0, A_PAD - 1)
    d1_ref[...] = jnp.clip(d1, 0, A_PAD - 1)
    g_i = jax.lax.broadcasted_iota(jnp.int32, (G_MAX, E), 0)
    be = jnp.sum((jnp.broadcast_to(bcum, (G_MAX, E)) <= g_i).astype(jnp.int32),
                 axis=1, keepdims=True)
    be_ref[...] = jnp.clip(be, 0, E - 1)
    nb_ref[...] = bcum[:, E - 1:E]


def _schedule(i0, i1, r0, r1, cnt):
    full = lambda shape: pl.BlockSpec(shape, lambda: tuple(0 for _ in shape))
    return pl.pallas_call(
        _sched_body,
        grid=(),
        in_specs=[full((S, 1)), full((S, 1)), full((S, 1)), full((S, 1)),
                  full((1, E))],
        out_specs=[full((S, 1)), full((S, 1)), full((G_MAX, 1)), full((1, 1))],
        out_shape=[
            jax.ShapeDtypeStruct((S, 1), jnp.int32),
            jax.ShapeDtypeStruct((S, 1), jnp.int32),
            jax.ShapeDtypeStruct((G_MAX, 1), jnp.int32),
            jax.ShapeDtypeStruct((1, 1), jnp.int32),
        ],
    )(i0, i1, r0, r1, cnt)


def _sc_dispatch(h2, d0f, d1f):
    mesh = plsc.VectorSubcoreMesh(core_axis_name="c", subcore_axis_name="s")

    @functools.partial(
        pl.kernel, mesh=mesh,
        out_type=jax.ShapeDtypeStruct((A_PAD, D), jnp.float32),
        scratch_types=[
            pltpu.VMEM((CHUNK,), jnp.int32),
            pltpu.VMEM((CHUNK, D), jnp.float32),
            pltpu.SemaphoreType.DMA,
        ],
    )
    def body(h2_hbm, d0_hbm, d1_hbm, x_hbm, idx_v, rows_v, sem):
        wid = lax.axis_index("s") * 2 + lax.axis_index("c")
        base = wid * CHUNK
        pltpu.sync_copy(h2_hbm.at[pl.ds(base, CHUNK)], rows_v)
        pltpu.sync_copy(d0_hbm.at[pl.ds(base, CHUNK)], idx_v)
        pltpu.async_copy(rows_v, x_hbm.at[idx_v], sem).wait()
        pltpu.sync_copy(d1_hbm.at[pl.ds(base, CHUNK)], idx_v)
        pltpu.async_copy(rows_v, x_hbm.at[idx_v], sem).wait()

    return body(h2, d0f, d1f)


def _sc_combine(ys, d0f, d1f):
    mesh = plsc.VectorSubcoreMesh(core_axis_name="c", subcore_axis_name="s")

    @functools.partial(
        pl.kernel, mesh=mesh,
        out_type=[
            jax.ShapeDtypeStruct((S, D), jnp.float32),
            jax.ShapeDtypeStruct((S, D), jnp.float32),
        ],
        scratch_types=[
            pltpu.VMEM((CHUNK,), jnp.int32),
            pltpu.VMEM((CHUNK, D), jnp.float32),
            pltpu.SemaphoreType.DMA,
        ],
    )
    def body(y_hbm, d0_hbm, d1_hbm, y0_hbm, y1_hbm, idx_v, buf_v, sem):
        wid = lax.axis_index("s") * 2 + lax.axis_index("c")
        base = wid * CHUNK
        pltpu.sync_copy(d0_hbm.at[pl.ds(base, CHUNK)], idx_v)
        pltpu.async_copy(y_hbm.at[idx_v], buf_v, sem).wait()
        pltpu.sync_copy(buf_v, y0_hbm.at[pl.ds(base, CHUNK)])
        pltpu.sync_copy(d1_hbm.at[pl.ds(base, CHUNK)], idx_v)
        pltpu.async_copy(y_hbm.at[idx_v], buf_v, sem).wait()
        pltpu.sync_copy(buf_v, y1_hbm.at[pl.ds(base, CHUNK)])

    return body(ys, d0f, d1f)


def _ffn_body(be_ref, nb_ref, x_ref, w1_ref, b1_ref, w2_ref, b2_ref, y_ref):
    g = pl.program_id(0)

    @pl.when(g < nb_ref[0])
    def _():
        xb = x_ref[...].astype(jnp.bfloat16)
        h = jnp.dot(xb, w1_ref[0], preferred_element_type=jnp.float32)
        h = jnp.maximum(h + b1_ref[0], 0.0).astype(jnp.bfloat16)
        y = jnp.dot(h, w2_ref[0], preferred_element_type=jnp.float32)
        y_ref[...] = y + b2_ref[0]


def _ffn(be, nb, xs, w1b, b1, w2b, b2):
    grid_spec = pltpu.PrefetchScalarGridSpec(
        num_scalar_prefetch=2,
        grid=(G_MAX,),
        in_specs=[
            pl.BlockSpec((FT, D), lambda g, be, nb: (g, 0)),
            pl.BlockSpec((1, D, DFF), lambda g, be, nb: (be[g], 0, 0)),
            pl.BlockSpec((1, 1, DFF), lambda g, be, nb: (be[g], 0, 0)),
            pl.BlockSpec((1, DFF, D), lambda g, be, nb: (be[g], 0, 0)),
            pl.BlockSpec((1, 1, D), lambda g, be, nb: (be[g], 0, 0)),
        ],
        out_specs=pl.BlockSpec((FT, D), lambda g, be, nb: (g, 0)),
    )
    return pl.pallas_call(
        _ffn_body,
        grid_spec=grid_spec,
        out_shape=jax.ShapeDtypeStruct((A_PAD, D), jnp.float32),
        compiler_params=pltpu.CompilerParams(
            dimension_semantics=("arbitrary",)),
    )(be, nb, xs, w1b, b1, w2b, b2)


def _combine_body(xa_ref, y0_ref, y1_ref, g0_ref, g1_ref, o_ref):
    o_ref[...] = (xa_ref[...] + g0_ref[...] * y0_ref[...]
                  + g1_ref[...] * y1_ref[...])


def _combine(xa, y0, y1, g0, g1):
    row = lambda: pl.BlockSpec((TT, D), lambda t: (t, 0))
    col = lambda: pl.BlockSpec((TT, 1), lambda t: (t, 0))
    return pl.pallas_call(
        _combine_body,
        grid=(NT,),
        in_specs=[row(), row(), row(), col(), col()],
        out_specs=row(),
        out_shape=jax.ShapeDtypeStruct((S, D), jnp.float32),
    )(xa, y0, y1, g0, g1)


def kernel(x, params):
    p = params
    x2d = x[0]
    wq_r = p['wq'].reshape(D, NH, HD).transpose(1, 0, 2)
    wk_r = p['wk'].reshape(D, NH, HD).transpose(1, 0, 2)
    wv_r = p['wv'].reshape(D, NH, HD).transpose(1, 0, 2)
    wqkv = jnp.concatenate([wq_r, wk_r, wv_r], axis=2)
    wqkv = wqkv.reshape(NH // 2, 2, D, 3 * HD).transpose(0, 2, 1, 3)
    wqkv = wqkv.reshape(NH // 2, D, 6 * HD)
    bq_r = p['bq'].reshape(NH, 1, HD)
    bk_r = p['bk'].reshape(NH, 1, HD)
    bv_r = p['bv'].reshape(NH, 1, HD)
    bqkv = jnp.concatenate([bq_r, bk_r, bv_r], axis=2)
    bqkv = bqkv.reshape(NH // 2, 1, 6 * HD)
    wo_r = p['wo']
    nz = jax.random.normal(jax.random.key(42), (B, S, E), jnp.float32)[0]
    w1b = p['e_w1'].astype(jnp.bfloat16)
    w2b = p['e_w2'].astype(jnp.bfloat16)
    b1r = p['e_b1'].reshape(E, 1, DFF)
    b2r = p['e_b2'].reshape(E, 1, D)

    h1 = _ln1(x2d, p['ln1_g'], p['ln1_b'])
    ctx = _attention(h1, wqkv, bqkv)
    xa, h2, i0, i1, r0, r1, g0, g1, cnt = _router(
        x2d, ctx, wo_r, p['bo'], p['ln2_g'], p['ln2_b'],
        p['router_w'], p['router_b'], p['noise_w'], p['noise_b'], nz)
    d0, d1, be, nb = _schedule(i0, i1, r0, r1, cnt)
    d0f = d0.reshape(S)
    d1f = d1.reshape(S)
    xs = _sc_dispatch(h2, d0f, d1f)
    ys = _ffn(be.reshape(G_MAX), nb.reshape(1), xs, w1b, b1r, w2b, b2r)
    y0, y1 = _sc_combine(ys, d0f, d1f)
    out = _combine(xa, y0, y1, g0, g1)
    return out.reshape(B, S, D)


# speedup vs baseline: 1.1792x; 1.1792x over previous
import functools
import math

import jax
import jax.numpy as jnp
from jax import lax
from jax.experimental import pallas as pl
from jax.experimental.pallas import tpu as pltpu
from jax.experimental.pallas import tpu_sc as plsc

B, S, D = 1, 2048, 768
NH, HD = 12, 64
E, TOPK = 8, 2
DFF = 4 * D
TT = 256
NT = S // TT
FT = 256
G_MAX = (S * TOPK) // FT + E
A_PAD = G_MAX * FT
NW = 32
CHUNK = S // NW
NEG_INF = float('-inf')


def _ln(x, g, b, eps=1e-5):
    mu = jnp.mean(x, axis=-1, keepdims=True)
    var = jnp.mean((x - mu) ** 2, axis=-1, keepdims=True)
    return (x - mu) * jax.lax.rsqrt(var + eps) * g + b


def _ln1_body(x_ref, g_ref, b_ref, o_ref):
    o_ref[...] = _ln(x_ref[...], g_ref[...], b_ref[...])


def _ln1(x2d, g, b):
    return pl.pallas_call(
        _ln1_body,
        grid=(NT,),
        in_specs=[
            pl.BlockSpec((TT, D), lambda t: (t, 0)),
            pl.BlockSpec((1, D), lambda t: (0, 0)),
            pl.BlockSpec((1, D), lambda t: (0, 0)),
        ],
        out_specs=pl.BlockSpec((TT, D), lambda t: (t, 0)),
        out_shape=jax.ShapeDtypeStruct((S, D), jnp.float32),
    )(x2d, g.reshape(1, D), b.reshape(1, D))


def _attn_body(x_ref, g1_ref, b1_ref, wqkv_ref, bqkv_ref, o_ref, h1_ref):
    @pl.when(pl.program_id(0) == 0)
    def _():
        h1_ref[...] = _ln(x_ref[...], g1_ref[...],
                          b1_ref[...]).astype(jnp.bfloat16)

    h1 = h1_ref[...]
    qkv = jnp.dot(h1, wqkv_ref[0],
                  preferred_element_type=jnp.float32) + bqkv_ref[0]
    outs = []
    for j in range(2):
        base = j * 3 * HD
        q = qkv[:, base:base + HD] * jnp.float32(1.0 / math.sqrt(HD))
        k = qkv[:, base + HD:base + 2 * HD]
        v = qkv[:, base + 2 * HD:base + 3 * HD]
        s = jax.lax.dot_general(q.astype(jnp.bfloat16),
                                k.astype(jnp.bfloat16),
                                (((1,), (1,)), ((), ())),
                                preferred_element_type=jnp.float32)
        m = jnp.max(s, axis=1, keepdims=True)
        p = jnp.exp(s - m)
        l = jnp.sum(p, axis=1, keepdims=True)
        ctx = jnp.dot(p.astype(jnp.bfloat16), v.astype(jnp.bfloat16),
                      preferred_element_type=jnp.float32)
        outs.append(ctx / l)
    o_ref[...] = jnp.concatenate(outs, axis=1)


def _attention(x2d, g1, b1, wqkv, bqkv):
    return pl.pallas_call(
        _attn_body,
        grid=(NH // 2,),
        in_specs=[
            pl.BlockSpec((S, D), lambda h: (0, 0)),
            pl.BlockSpec((1, D), lambda h: (0, 0)),
            pl.BlockSpec((1, D), lambda h: (0, 0)),
            pl.BlockSpec((1, D, 6 * HD), lambda h: (h, 0, 0)),
            pl.BlockSpec((1, 1, 6 * HD), lambda h: (h, 0, 0)),
        ],
        out_specs=pl.BlockSpec((S, 2 * HD), lambda h: (0, h)),
        out_shape=jax.ShapeDtypeStruct((S, D), jnp.float32),
        scratch_shapes=[pltpu.VMEM((S, D), jnp.bfloat16)],
        compiler_params=pltpu.CompilerParams(
            dimension_semantics=("arbitrary",)),
    )(x2d, g1.reshape(1, D), b1.reshape(1, D), wqkv, bqkv)


def _router_body(x_ref, ctx_ref, wo_ref, bo_ref, g2_ref, b2_ref,
                 rw_ref, rb_ref, nw_ref, nb_ref, nz_ref,
                 xa_ref, h2_ref, i0_ref, i1_ref, r0_ref, r1_ref,
                 g0_ref, g1_ref, cnt_ref, carry_ref):
    t = pl.program_id(0)
    sa = jnp.dot(ctx_ref[...].astype(jnp.bfloat16),
                 wo_ref[...].astype(jnp.bfloat16),
                 preferred_element_type=jnp.float32) + bo_ref[...]
    xa = x_ref[...] + sa
    xa_ref[...] = xa
    h2 = _ln(xa, g2_ref[...], b2_ref[...])
    h2_ref[...] = h2
    h2b = h2.astype(jnp.bfloat16)
    logits = jnp.dot(h2b, rw_ref[...].astype(jnp.bfloat16),
                     preferred_element_type=jnp.float32) + rb_ref[...]
    nl = jnp.dot(h2b, nw_ref[...].astype(jnp.bfloat16),
                 preferred_element_type=jnp.float32) + nb_ref[...]
    sp = jnp.maximum(nl, 0.0) + jnp.log1p(jnp.exp(-jnp.abs(nl)))
    lg = logits + nz_ref[...] * sp
    cols = jax.lax.broadcasted_iota(jnp.int32, (TT, E), 1)
    v0 = jnp.max(lg, axis=1, keepdims=True)
    i0 = jnp.argmax(lg, axis=1).reshape(TT, 1)
    masked = jnp.where(cols == i0, NEG_INF, lg)
    v1 = jnp.max(masked, axis=1, keepdims=True)
    i1 = jnp.argmax(masked, axis=1).reshape(TT, 1)
    ex = jnp.exp(v1 - v0)
    den = 1.0 + ex
    i0_ref[...] = i0
    i1_ref[...] = i1
    g0_ref[...] = 1.0 / den
    g1_ref[...] = ex / den

    oh0 = (cols == i0).astype(jnp.float32)
    oh1 = (cols == i1).astype(jnp.float32)
    m = oh0 + oh1
    rows_i = jax.lax.broadcasted_iota(jnp.int32, (TT, TT), 0)
    cols_i = jax.lax.broadcasted_iota(jnp.int32, (TT, TT), 1)
    tri = (cols_i < rows_i).astype(jnp.bfloat16)
    csum = jnp.dot(tri, m.astype(jnp.bfloat16),
                   preferred_element_type=jnp.float32)

    @pl.when(t == 0)
    def _():
        carry_ref[...] = jnp.zeros((1, E), jnp.float32)

    carry = carry_ref[...]
    rank = carry + csum
    r0_ref[...] = jnp.sum(oh0 * rank, axis=1, keepdims=True).astype(jnp.int32)
    r1_ref[...] = jnp.sum(oh1 * rank, axis=1, keepdims=True).astype(jnp.int32)
    new_carry = carry + jnp.sum(m, axis=0, keepdims=True)
    carry_ref[...] = new_carry
    cnt_ref[...] = new_carry.astype(jnp.int32)


def _router(x2d, ctx, wo_r, bo, g2, b2, rw, rb, nw, nb, nz):
    fixed = lambda shape: pl.BlockSpec(shape, lambda t: tuple(0 for _ in shape))
    col = lambda: pl.BlockSpec((TT, 1), lambda t: (t, 0))
    return pl.pallas_call(
        _router_body,
        grid=(NT,),
        in_specs=[
            pl.BlockSpec((TT, D), lambda t: (t, 0)),
            pl.BlockSpec((TT, D), lambda t: (t, 0)),
            fixed((D, D)), fixed((1, D)),
            fixed((1, D)), fixed((1, D)),
            fixed((D, E)), fixed((1, E)), fixed((D, E)), fixed((1, E)),
            pl.BlockSpec((TT, E), lambda t: (t, 0)),
        ],
        out_specs=[
            pl.BlockSpec((TT, D), lambda t: (t, 0)),
            pl.BlockSpec((TT, D), lambda t: (t, 0)),
            col(), col(), col(), col(), col(), col(),
            pl.BlockSpec((1, E), lambda t: (0, 0)),
        ],
        out_shape=[
            jax.ShapeDtypeStruct((S, D), jnp.float32),
            jax.ShapeDtypeStruct((S, D), jnp.float32),
            jax.ShapeDtypeStruct((S, 1), jnp.int32),
            jax.ShapeDtypeStruct((S, 1), jnp.int32),
            jax.ShapeDtypeStruct((S, 1), jnp.int32),
            jax.ShapeDtypeStruct((S, 1), jnp.int32),
            jax.ShapeDtypeStruct((S, 1), jnp.float32),
            jax.ShapeDtypeStruct((S, 1), jnp.float32),
            jax.ShapeDtypeStruct((1, E), jnp.int32),
        ],
        scratch_shapes=[pltpu.VMEM((1, E), jnp.float32)],
        compiler_params=pltpu.CompilerParams(
            dimension_semantics=("arbitrary",)),
    )(x2d, ctx, wo_r, bo.reshape(1, D), g2.reshape(1, D), b2.reshape(1, D),
      rw, rb.reshape(1, E), nw, nb.reshape(1, E), nz)


def _sched_body(i0_ref, i1_ref, r0_ref, r1_ref, cnt_ref,
                d0_ref, d1_ref, be_ref, nb_ref):
    cnt = cnt_ref[...]
    nbe = (cnt + (FT - 1)) // FT
    ecols = jax.lax.broadcasted_iota(jnp.int32, (E, E), 1)
    erows = jax.lax.broadcasted_iota(jnp.int32, (E, E), 0)
    triu = (erows <= ecols).astype(jnp.bfloat16)
    bcum = jnp.dot(nbe.astype(jnp.bfloat16), triu,
                   preferred_element_type=jnp.float32).astype(jnp.int32)
    off = (bcum - nbe) * FT
    cols = jax.lax.broadcasted_iota(jnp.int32, (S, E), 1)
    oh0 = (cols == i0_ref[...]).astype(jnp.float32)
    oh1 = (cols == i1_ref[...]).astype(jnp.float32)
    offf = off.astype(jnp.float32)
    d0 = r0_ref[...] + jnp.sum(oh0 * offf, axis=1,
                               keepdims=True).astype(jnp.int32)
    d1 = r1_ref[...] + jnp.sum(oh1 * offf, axis=1,
                               keepdims=True).astype(jnp.int32)
    d0_ref[...] = jnp.clip(d0, 0, A_PAD - 1)
    d1_ref[...] = jnp.clip(d1, 0, A_PAD - 1)
    g_i = jax.lax.broadcasted_iota(jnp.int32, (G_MAX, E), 0)
    be = jnp.sum((jnp.broadcast_to(bcum, (G_MAX, E)) <= g_i).astype(jnp.int32),
                 axis=1, keepdims=True)
    be_ref[...] = jnp.clip(be, 0, E - 1)
    nb_ref[...] = bcum[:, E - 1:E]


def _schedule(i0, i1, r0, r1, cnt):
    full = lambda shape: pl.BlockSpec(shape, lambda: tuple(0 for _ in shape))
    return pl.pallas_call(
        _sched_body,
        grid=(),
        in_specs=[full((S, 1)), full((S, 1)), full((S, 1)), full((S, 1)),
                  full((1, E))],
        out_specs=[full((S, 1)), full((S, 1)), full((G_MAX, 1)), full((1, 1))],
        out_shape=[
            jax.ShapeDtypeStruct((S, 1), jnp.int32),
            jax.ShapeDtypeStruct((S, 1), jnp.int32),
            jax.ShapeDtypeStruct((G_MAX, 1), jnp.int32),
            jax.ShapeDtypeStruct((1, 1), jnp.int32),
        ],
    )(i0, i1, r0, r1, cnt)


def _sc_dispatch(h2, d0f, d1f):
    mesh = plsc.VectorSubcoreMesh(core_axis_name="c", subcore_axis_name="s")

    @functools.partial(
        pl.kernel, mesh=mesh,
        out_type=jax.ShapeDtypeStruct((A_PAD, D), jnp.float32),
        scratch_types=[
            pltpu.VMEM((CHUNK,), jnp.int32),
            pltpu.VMEM((CHUNK, D), jnp.float32),
            pltpu.SemaphoreType.DMA,
        ],
    )
    def body(h2_hbm, d0_hbm, d1_hbm, x_hbm, idx_v, rows_v, sem):
        wid = lax.axis_index("s") * 2 + lax.axis_index("c")
        base = wid * CHUNK
        pltpu.sync_copy(h2_hbm.at[pl.ds(base, CHUNK)], rows_v)
        pltpu.sync_copy(d0_hbm.at[pl.ds(base, CHUNK)], idx_v)
        pltpu.async_copy(rows_v, x_hbm.at[idx_v], sem).wait()
        pltpu.sync_copy(d1_hbm.at[pl.ds(base, CHUNK)], idx_v)
        pltpu.async_copy(rows_v, x_hbm.at[idx_v], sem).wait()

    return body(h2, d0f, d1f)


def _sc_combine(ys, d0f, d1f):
    mesh = plsc.VectorSubcoreMesh(core_axis_name="c", subcore_axis_name="s")

    @functools.partial(
        pl.kernel, mesh=mesh,
        out_type=[
            jax.ShapeDtypeStruct((S, D), jnp.float32),
            jax.ShapeDtypeStruct((S, D), jnp.float32),
        ],
        scratch_types=[
            pltpu.VMEM((CHUNK,), jnp.int32),
            pltpu.VMEM((CHUNK, D), jnp.float32),
            pltpu.SemaphoreType.DMA,
        ],
    )
    def body(y_hbm, d0_hbm, d1_hbm, y0_hbm, y1_hbm, idx_v, buf_v, sem):
        wid = lax.axis_index("s") * 2 + lax.axis_index("c")
        base = wid * CHUNK
        pltpu.sync_copy(d0_hbm.at[pl.ds(base, CHUNK)], idx_v)
        pltpu.async_copy(y_hbm.at[idx_v], buf_v, sem).wait()
        pltpu.sync_copy(buf_v, y0_hbm.at[pl.ds(base, CHUNK)])
        pltpu.sync_copy(d1_hbm.at[pl.ds(base, CHUNK)], idx_v)
        pltpu.async_copy(y_hbm.at[idx_v], buf_v, sem).wait()
        pltpu.sync_copy(buf_v, y1_hbm.at[pl.ds(base, CHUNK)])

    return body(ys, d0f, d1f)


def _ffn_body(be_ref, nb_ref, x_ref, w1_ref, b1_ref, w2_ref, b2_ref, y_ref):
    g = pl.program_id(0)

    @pl.when(g < nb_ref[0])
    def _():
        xb = x_ref[...].astype(jnp.bfloat16)
        h = jnp.dot(xb, w1_ref[0].astype(jnp.bfloat16),
                    preferred_element_type=jnp.float32)
        h = jnp.maximum(h + b1_ref[0], 0.0).astype(jnp.bfloat16)
        y = jnp.dot(h, w2_ref[0].astype(jnp.bfloat16),
                    preferred_element_type=jnp.float32)
        y_ref[...] = y + b2_ref[0]


def _ffn(be, nb, xs, w1b, b1, w2b, b2):
    grid_spec = pltpu.PrefetchScalarGridSpec(
        num_scalar_prefetch=2,
        grid=(G_MAX,),
        in_specs=[
            pl.BlockSpec((FT, D), lambda g, be, nb: (g, 0)),
            pl.BlockSpec((1, D, DFF), lambda g, be, nb: (be[g], 0, 0)),
            pl.BlockSpec((1, 1, DFF), lambda g, be, nb: (be[g], 0, 0)),
            pl.BlockSpec((1, DFF, D), lambda g, be, nb: (be[g], 0, 0)),
            pl.BlockSpec((1, 1, D), lambda g, be, nb: (be[g], 0, 0)),
        ],
        out_specs=pl.BlockSpec((FT, D), lambda g, be, nb: (g, 0)),
    )
    return pl.pallas_call(
        _ffn_body,
        grid_spec=grid_spec,
        out_shape=jax.ShapeDtypeStruct((A_PAD, D), jnp.float32),
        compiler_params=pltpu.CompilerParams(
            dimension_semantics=("arbitrary",)),
    )(be, nb, xs, w1b, b1, w2b, b2)


def _combine_body(xa_ref, y0_ref, y1_ref, g0_ref, g1_ref, o_ref):
    o_ref[...] = (xa_ref[...] + g0_ref[...] * y0_ref[...]
                  + g1_ref[...] * y1_ref[...])


def _combine(xa, y0, y1, g0, g1):
    row = lambda: pl.BlockSpec((TT, D), lambda t: (t, 0))
    col = lambda: pl.BlockSpec((TT, 1), lambda t: (t, 0))
    return pl.pallas_call(
        _combine_body,
        grid=(NT,),
        in_specs=[row(), row(), row(), col(), col()],
        out_specs=row(),
        out_shape=jax.ShapeDtypeStruct((S, D), jnp.float32),
    )(xa, y0, y1, g0, g1)


def kernel(x, params):
    p = params
    x2d = x[0]
    wq_r = p['wq'].reshape(D, NH, HD).transpose(1, 0, 2)
    wk_r = p['wk'].reshape(D, NH, HD).transpose(1, 0, 2)
    wv_r = p['wv'].reshape(D, NH, HD).transpose(1, 0, 2)
    wqkv = jnp.concatenate([wq_r, wk_r, wv_r], axis=2)
    wqkv = wqkv.reshape(NH // 2, 2, D, 3 * HD).transpose(0, 2, 1, 3)
    wqkv = wqkv.reshape(NH // 2, D, 6 * HD)
    bq_r = p['bq'].reshape(NH, 1, HD)
    bk_r = p['bk'].reshape(NH, 1, HD)
    bv_r = p['bv'].reshape(NH, 1, HD)
    bqkv = jnp.concatenate([bq_r, bk_r, bv_r], axis=2)
    bqkv = bqkv.reshape(NH // 2, 1, 6 * HD)
    wo_r = p['wo']
    nz = jax.random.normal(jax.random.key(42), (B, S, E), jnp.float32)[0]
    b1r = p['e_b1'].reshape(E, 1, DFF)
    b2r = p['e_b2'].reshape(E, 1, D)

    ctx = _attention(x2d, p['ln1_g'], p['ln1_b'], wqkv, bqkv)
    xa, h2, i0, i1, r0, r1, g0, g1, cnt = _router(
        x2d, ctx, wo_r, p['bo'], p['ln2_g'], p['ln2_b'],
        p['router_w'], p['router_b'], p['noise_w'], p['noise_b'], nz)
    d0, d1, be, nb = _schedule(i0, i1, r0, r1, cnt)
    d0f = d0.reshape(S)
    d1f = d1.reshape(S)
    xs = _sc_dispatch(h2, d0f, d1f)
    ys = _ffn(be.reshape(G_MAX), nb.reshape(1), xs, p['e_w1'], b1r, p['e_w2'], b2r)
    y0, y1 = _sc_combine(ys, d0f, d1f)
    out = _combine(xa, y0, y1, g0, g1)
    return out.reshape(B, S, D)


# speedup vs baseline: 1.1971x; 1.0152x over previous
import functools
import math

import jax
import jax.numpy as jnp
from jax import lax
from jax.experimental import pallas as pl
from jax.experimental.pallas import tpu as pltpu
from jax.experimental.pallas import tpu_sc as plsc

B, S, D = 1, 2048, 768
NH, HD = 12, 64
E, TOPK = 8, 2
DFF = 4 * D
TT = 512
NT = S // TT
FT = 256
G_MAX = (S * TOPK) // FT + E
A_PAD = G_MAX * FT
NW = 32
CHUNK = S // NW
NEG_INF = float('-inf')


def _ln(x, g, b, eps=1e-5):
    mu = jnp.mean(x, axis=-1, keepdims=True)
    var = jnp.mean((x - mu) ** 2, axis=-1, keepdims=True)
    return (x - mu) * jax.lax.rsqrt(var + eps) * g + b


def _attn_body(x_ref, g1_ref, b1_ref, wqkv_ref, bqkv_ref, o_ref, h1_ref):
    @pl.when(pl.program_id(0) == 0)
    def _():
        h1_ref[...] = _ln(x_ref[...], g1_ref[...],
                          b1_ref[...]).astype(jnp.bfloat16)

    h1 = h1_ref[...]
    qkv = jnp.dot(h1, wqkv_ref[0],
                  preferred_element_type=jnp.float32) + bqkv_ref[0]
    outs = []
    for j in range(2):
        base = j * 3 * HD
        q = qkv[:, base:base + HD] * jnp.float32(1.0 / math.sqrt(HD))
        k = qkv[:, base + HD:base + 2 * HD]
        v = qkv[:, base + 2 * HD:base + 3 * HD]
        s = jax.lax.dot_general(q.astype(jnp.bfloat16),
                                k.astype(jnp.bfloat16),
                                (((1,), (1,)), ((), ())),
                                preferred_element_type=jnp.float32)
        m = jnp.max(s, axis=1, keepdims=True)
        p = jnp.exp(s - m)
        l = jnp.sum(p, axis=1, keepdims=True)
        ctx = jnp.dot(p.astype(jnp.bfloat16), v.astype(jnp.bfloat16),
                      preferred_element_type=jnp.float32)
        outs.append(ctx / l)
    o_ref[...] = jnp.concatenate(outs, axis=1)


def _attention(x2d, g1, b1, wqkv, bqkv):
    return pl.pallas_call(
        _attn_body,
        grid=(NH // 2,),
        in_specs=[
            pl.BlockSpec((S, D), lambda h: (0, 0)),
            pl.BlockSpec((1, D), lambda h: (0, 0)),
            pl.BlockSpec((1, D), lambda h: (0, 0)),
            pl.BlockSpec((1, D, 6 * HD), lambda h: (h, 0, 0)),
            pl.BlockSpec((1, 1, 6 * HD), lambda h: (h, 0, 0)),
        ],
        out_specs=pl.BlockSpec((S, 2 * HD), lambda h: (0, h)),
        out_shape=jax.ShapeDtypeStruct((S, D), jnp.float32),
        scratch_shapes=[pltpu.VMEM((S, D), jnp.bfloat16)],
        compiler_params=pltpu.CompilerParams(
            dimension_semantics=("arbitrary",)),
    )(x2d, g1.reshape(1, D), b1.reshape(1, D), wqkv, bqkv)


def _router_body(x_ref, ctx_ref, wo_ref, bo_ref, g2_ref, b2_ref,
                 rw_ref, rb_ref, nw_ref, nb_ref, nz_ref,
                 xa_ref, h2_ref, i0_ref, i1_ref, r0_ref, r1_ref,
                 g0_ref, g1_ref, cnt_ref, carry_ref):
    t = pl.program_id(0)
    sa = jnp.dot(ctx_ref[...].astype(jnp.bfloat16),
                 wo_ref[...].astype(jnp.bfloat16),
                 preferred_element_type=jnp.float32) + bo_ref[...]
    xa = x_ref[...] + sa
    xa_ref[...] = xa
    h2 = _ln(xa, g2_ref[...], b2_ref[...])
    h2_ref[...] = h2
    h2b = h2.astype(jnp.bfloat16)
    logits = jnp.dot(h2b, rw_ref[...].astype(jnp.bfloat16),
                     preferred_element_type=jnp.float32) + rb_ref[...]
    nl = jnp.dot(h2b, nw_ref[...].astype(jnp.bfloat16),
                 preferred_element_type=jnp.float32) + nb_ref[...]
    sp = jnp.maximum(nl, 0.0) + jnp.log1p(jnp.exp(-jnp.abs(nl)))
    lg = logits + nz_ref[...] * sp
    cols = jax.lax.broadcasted_iota(jnp.int32, (TT, E), 1)
    v0 = jnp.max(lg, axis=1, keepdims=True)
    i0 = jnp.argmax(lg, axis=1).reshape(TT, 1)
    masked = jnp.where(cols == i0, NEG_INF, lg)
    v1 = jnp.max(masked, axis=1, keepdims=True)
    i1 = jnp.argmax(masked, axis=1).reshape(TT, 1)
    ex = jnp.exp(v1 - v0)
    den = 1.0 + ex
    i0_ref[...] = i0
    i1_ref[...] = i1
    g0_ref[...] = 1.0 / den
    g1_ref[...] = ex / den

    oh0 = (cols == i0).astype(jnp.float32)
    oh1 = (cols == i1).astype(jnp.float32)
    m = oh0 + oh1
    rows_i = jax.lax.broadcasted_iota(jnp.int32, (TT, TT), 0)
    cols_i = jax.lax.broadcasted_iota(jnp.int32, (TT, TT), 1)
    tri = (cols_i < rows_i).astype(jnp.bfloat16)
    csum = jnp.dot(tri, m.astype(jnp.bfloat16),
                   preferred_element_type=jnp.float32)

    @pl.when(t == 0)
    def _():
        carry_ref[...] = jnp.zeros((1, E), jnp.float32)

    carry = carry_ref[...]
    rank = carry + csum
    r0_ref[...] = jnp.sum(oh0 * rank, axis=1, keepdims=True).astype(jnp.int32)
    r1_ref[...] = jnp.sum(oh1 * rank, axis=1, keepdims=True).astype(jnp.int32)
    new_carry = carry + jnp.sum(m, axis=0, keepdims=True)
    carry_ref[...] = new_carry
    cnt_ref[...] = new_carry.astype(jnp.int32)


def _router(x2d, ctx, wo_r, bo, g2, b2, rw, rb, nw, nb, nz):
    fixed = lambda shape: pl.BlockSpec(shape, lambda t: tuple(0 for _ in shape))
    col = lambda: pl.BlockSpec((TT, 1), lambda t: (t, 0))
    return pl.pallas_call(
        _router_body,
        grid=(NT,),
        in_specs=[
            pl.BlockSpec((TT, D), lambda t: (t, 0)),
            pl.BlockSpec((TT, D), lambda t: (t, 0)),
            fixed((D, D)), fixed((1, D)),
            fixed((1, D)), fixed((1, D)),
            fixed((D, E)), fixed((1, E)), fixed((D, E)), fixed((1, E)),
            pl.BlockSpec((TT, E), lambda t: (t, 0)),
        ],
        out_specs=[
            pl.BlockSpec((TT, D), lambda t: (t, 0)),
            pl.BlockSpec((TT, D), lambda t: (t, 0)),
            col(), col(), col(), col(), col(), col(),
            pl.BlockSpec((1, E), lambda t: (0, 0)),
        ],
        out_shape=[
            jax.ShapeDtypeStruct((S, D), jnp.float32),
            jax.ShapeDtypeStruct((S, D), jnp.float32),
            jax.ShapeDtypeStruct((S, 1), jnp.int32),
            jax.ShapeDtypeStruct((S, 1), jnp.int32),
            jax.ShapeDtypeStruct((S, 1), jnp.int32),
            jax.ShapeDtypeStruct((S, 1), jnp.int32),
            jax.ShapeDtypeStruct((S, 1), jnp.float32),
            jax.ShapeDtypeStruct((S, 1), jnp.float32),
            jax.ShapeDtypeStruct((1, E), jnp.int32),
        ],
        scratch_shapes=[pltpu.VMEM((1, E), jnp.float32)],
        compiler_params=pltpu.CompilerParams(
            dimension_semantics=("arbitrary",)),
    )(x2d, ctx, wo_r, bo.reshape(1, D), g2.reshape(1, D), b2.reshape(1, D),
      rw, rb.reshape(1, E), nw, nb.reshape(1, E), nz)


def _sched_body(i0_ref, i1_ref, r0_ref, r1_ref, cnt_ref,
                d0_ref, d1_ref, be_ref, nb_ref):
    cnt = cnt_ref[...]
    nbe = (cnt + (FT - 1)) // FT
    ecols = jax.lax.broadcasted_iota(jnp.int32, (E, E), 1)
    erows = jax.lax.broadcasted_iota(jnp.int32, (E, E), 0)
    triu = (erows <= ecols).astype(jnp.bfloat16)
    bcum = jnp.dot(nbe.astype(jnp.bfloat16), triu,
                   preferred_element_type=jnp.float32).astype(jnp.int32)
    off = (bcum - nbe) * FT
    cols = jax.lax.broadcasted_iota(jnp.int32, (S, E), 1)
    oh0 = (cols == i0_ref[...]).astype(jnp.float32)
    oh1 = (cols == i1_ref[...]).astype(jnp.float32)
    offf = off.astype(jnp.float32)
    d0 = r0_ref[...] + jnp.sum(oh0 * offf, axis=1,
                               keepdims=True).astype(jnp.int32)
    d1 = r1_ref[...] + jnp.sum(oh1 * offf, axis=1,
                               keepdims=True).astype(jnp.int32)
    d0_ref[...] = jnp.clip(d0, 0, A_PAD - 1)
    d1_ref[...] = jnp.clip(d1, 0, A_PAD - 1)
    g_i = jax.lax.broadcasted_iota(jnp.int32, (G_MAX, E), 0)
    be = jnp.sum((jnp.broadcast_to(bcum, (G_MAX, E)) <= g_i).astype(jnp.int32),
                 axis=1, keepdims=True)
    be_ref[...] = jnp.clip(be, 0, E - 1)
    nb_ref[...] = bcum[:, E - 1:E]


def _schedule(i0, i1, r0, r1, cnt):
    full = lambda shape: pl.BlockSpec(shape, lambda: tuple(0 for _ in shape))
    return pl.pallas_call(
        _sched_body,
        grid=(),
        in_specs=[full((S, 1)), full((S, 1)), full((S, 1)), full((S, 1)),
                  full((1, E))],
        out_specs=[full((S, 1)), full((S, 1)), full((G_MAX, 1)), full((1, 1))],
        out_shape=[
            jax.ShapeDtypeStruct((S, 1), jnp.int32),
            jax.ShapeDtypeStruct((S, 1), jnp.int32),
            jax.ShapeDtypeStruct((G_MAX, 1), jnp.int32),
            jax.ShapeDtypeStruct((1, 1), jnp.int32),
        ],
    )(i0, i1, r0, r1, cnt)


def _sc_dispatch(h2, d0f, d1f):
    mesh = plsc.VectorSubcoreMesh(core_axis_name="c", subcore_axis_name="s")

    @functools.partial(
        pl.kernel, mesh=mesh,
        out_type=jax.ShapeDtypeStruct((A_PAD, D), jnp.float32),
        scratch_types=[
            pltpu.VMEM((CHUNK,), jnp.int32),
            pltpu.VMEM((CHUNK, D), jnp.float32),
            pltpu.SemaphoreType.DMA,
        ],
    )
    def body(h2_hbm, d0_hbm, d1_hbm, x_hbm, idx_v, rows_v, sem):
        wid = lax.axis_index("s") * 2 + lax.axis_index("c")
        base = wid * CHUNK
        pltpu.sync_copy(h2_hbm.at[pl.ds(base, CHUNK)], rows_v)
        pltpu.sync_copy(d0_hbm.at[pl.ds(base, CHUNK)], idx_v)
        pltpu.async_copy(rows_v, x_hbm.at[idx_v], sem).wait()
        pltpu.sync_copy(d1_hbm.at[pl.ds(base, CHUNK)], idx_v)
        pltpu.async_copy(rows_v, x_hbm.at[idx_v], sem).wait()

    return body(h2, d0f, d1f)


def _sc_combine(ys, d0f, d1f):
    mesh = plsc.VectorSubcoreMesh(core_axis_name="c", subcore_axis_name="s")

    @functools.partial(
        pl.kernel, mesh=mesh,
        out_type=[
            jax.ShapeDtypeStruct((S, D), jnp.float32),
            jax.ShapeDtypeStruct((S, D), jnp.float32),
        ],
        scratch_types=[
            pltpu.VMEM((CHUNK,), jnp.int32),
            pltpu.VMEM((CHUNK, D), jnp.float32),
            pltpu.SemaphoreType.DMA,
        ],
    )
    def body(y_hbm, d0_hbm, d1_hbm, y0_hbm, y1_hbm, idx_v, buf_v, sem):
        wid = lax.axis_index("s") * 2 + lax.axis_index("c")
        base = wid * CHUNK
        pltpu.sync_copy(d0_hbm.at[pl.ds(base, CHUNK)], idx_v)
        pltpu.async_copy(y_hbm.at[idx_v], buf_v, sem).wait()
        pltpu.sync_copy(buf_v, y0_hbm.at[pl.ds(base, CHUNK)])
        pltpu.sync_copy(d1_hbm.at[pl.ds(base, CHUNK)], idx_v)
        pltpu.async_copy(y_hbm.at[idx_v], buf_v, sem).wait()
        pltpu.sync_copy(buf_v, y1_hbm.at[pl.ds(base, CHUNK)])

    return body(ys, d0f, d1f)


def _ffn_body(be_ref, nb_ref, x_ref, w1_ref, b1_ref, w2_ref, b2_ref, y_ref):
    g = pl.program_id(0)

    @pl.when(g < nb_ref[0])
    def _():
        xb = x_ref[...].astype(jnp.bfloat16)
        h = jnp.dot(xb, w1_ref[0].astype(jnp.bfloat16),
                    preferred_element_type=jnp.float32)
        h = jnp.maximum(h + b1_ref[0], 0.0).astype(jnp.bfloat16)
        y = jnp.dot(h, w2_ref[0].astype(jnp.bfloat16),
                    preferred_element_type=jnp.float32)
        y_ref[...] = y + b2_ref[0]


def _ffn(be, nb, xs, w1b, b1, w2b, b2):
    grid_spec = pltpu.PrefetchScalarGridSpec(
        num_scalar_prefetch=2,
        grid=(G_MAX,),
        in_specs=[
            pl.BlockSpec((FT, D), lambda g, be, nb: (g, 0)),
            pl.BlockSpec((1, D, DFF), lambda g, be, nb: (be[g], 0, 0)),
            pl.BlockSpec((1, 1, DFF), lambda g, be, nb: (be[g], 0, 0)),
            pl.BlockSpec((1, DFF, D), lambda g, be, nb: (be[g], 0, 0)),
            pl.BlockSpec((1, 1, D), lambda g, be, nb: (be[g], 0, 0)),
        ],
        out_specs=pl.BlockSpec((FT, D), lambda g, be, nb: (g, 0)),
    )
    return pl.pallas_call(
        _ffn_body,
        grid_spec=grid_spec,
        out_shape=jax.ShapeDtypeStruct((A_PAD, D), jnp.float32),
        compiler_params=pltpu.CompilerParams(
            dimension_semantics=("arbitrary",)),
    )(be, nb, xs, w1b, b1, w2b, b2)


def _combine_body(xa_ref, y0_ref, y1_ref, g0_ref, g1_ref, o_ref):
    o_ref[...] = (xa_ref[...] + g0_ref[...] * y0_ref[...]
                  + g1_ref[...] * y1_ref[...])


def _combine(xa, y0, y1, g0, g1):
    row = lambda: pl.BlockSpec((TT, D), lambda t: (t, 0))
    col = lambda: pl.BlockSpec((TT, 1), lambda t: (t, 0))
    return pl.pallas_call(
        _combine_body,
        grid=(NT,),
        in_specs=[row(), row(), row(), col(), col()],
        out_specs=row(),
        out_shape=jax.ShapeDtypeStruct((S, D), jnp.float32),
    )(xa, y0, y1, g0, g1)


def kernel(x, params):
    p = params
    x2d = x[0]
    wq_r = p['wq'].reshape(D, NH, HD).transpose(1, 0, 2)
    wk_r = p['wk'].reshape(D, NH, HD).transpose(1, 0, 2)
    wv_r = p['wv'].reshape(D, NH, HD).transpose(1, 0, 2)
    wqkv = jnp.concatenate([wq_r, wk_r, wv_r], axis=2)
    wqkv = wqkv.reshape(NH // 2, 2, D, 3 * HD).transpose(0, 2, 1, 3)
    wqkv = wqkv.reshape(NH // 2, D, 6 * HD)
    bq_r = p['bq'].reshape(NH, 1, HD)
    bk_r = p['bk'].reshape(NH, 1, HD)
    bv_r = p['bv'].reshape(NH, 1, HD)
    bqkv = jnp.concatenate([bq_r, bk_r, bv_r], axis=2)
    bqkv = bqkv.reshape(NH // 2, 1, 6 * HD)
    wo_r = p['wo']
    nz = jax.random.normal(jax.random.key(42), (B, S, E), jnp.float32)[0]
    b1r = p['e_b1'].reshape(E, 1, DFF)
    b2r = p['e_b2'].reshape(E, 1, D)

    ctx = _attention(x2d, p['ln1_g'], p['ln1_b'], wqkv, bqkv)
    xa, h2, i0, i1, r0, r1, g0, g1, cnt = _router(
        x2d, ctx, wo_r, p['bo'], p['ln2_g'], p['ln2_b'],
        p['router_w'], p['router_b'], p['noise_w'], p['noise_b'], nz)
    d0, d1, be, nb = _schedule(i0, i1, r0, r1, cnt)
    d0f = d0.reshape(S)
    d1f = d1.reshape(S)
    xs = _sc_dispatch(h2, d0f, d1f)
    ys = _ffn(be.reshape(G_MAX), nb.reshape(1), xs, p['e_w1'], b1r, p['e_w2'], b2r)
    y0, y1 = _sc_combine(ys, d0f, d1f)
    out = _combine(xa, y0, y1, g0, g1)
    return out.reshape(B, S, D)
